# baseline (speedup 1.0000x reference)
import functools

import jax
import jax.numpy as jnp
from jax import lax
from jax.experimental import pallas as pl
from jax.experimental.pallas import tpu as pltpu
from jax.experimental.pallas import tpu_sc as plsc

NC = 2
NS = 16
LANES = 128


def _deg_body(dst2, degp, acc, zbuf, ones, didx, sem, dsem0, dsem1):
    c = lax.axis_index("c")
    s = lax.axis_index("s")
    wid = s * NC + c
    acc_rows = acc.shape[0]
    zt = zbuf.shape[0]

    z16 = jnp.zeros((16,), jnp.float32)

    def zb(i, _):
        zbuf[pl.ds(i * 16, 16)] = z16
        return 0

    lax.fori_loop(0, zt // 16, zb, 0)

    def ob(i, _):
        ones[pl.ds(i * 16, 16)] = z16 + 1.0
        return 0

    lax.fori_loop(0, 128 // 16, ob, 0)

    pltpu.sync_copy(zbuf, acc.at[pl.ds(s * zt, zt)])
    plsc.subcore_barrier()

    rows_per_tile = dst2.shape[0] // (NC * NS)
    row0 = wid * rows_per_tile
    ng = rows_per_tile // 4

    pltpu.sync_copy(dst2.at[pl.ds(row0, 4)], didx.at[0])
    dsems = (dsem0, dsem1)

    def outer(g, _):
        p = lax.rem(g, 2)
        q = 1 - p

        @pl.when(g + 1 < ng)
        def _():
            pltpu.async_copy(dst2.at[pl.ds(row0 + (g + 1) * 4, 4)],
                             didx.at[q], sem)

        for j in range(4):
            if j >= 2:
                pltpu.make_async_copy(ones, acc.at[didx.at[p, j - 2]],
                                      dsems[j & 1]).wait()
            else:
                @pl.when(g > 0)
                def _():
                    pltpu.make_async_copy(
                        ones, acc.at[didx.at[lax.rem(g + 1, 2), j + 2]],
                        dsems[j & 1]).wait()
            pltpu.async_copy(ones, acc.at[didx.at[p, j]], dsems[j & 1],
                             add=True)

        @pl.when(g + 1 < ng)
        def _():
            pltpu.make_async_copy(dst2.at[pl.ds(0, 4)], didx.at[q],
                                  sem).wait()
        return 0

    lax.fori_loop(0, ng, outer, 0)
    pf = (ng - 1) % 2
    pltpu.make_async_copy(ones, acc.at[didx.at[pf, 2]], dsems[0]).wait()
    pltpu.make_async_copy(ones, acc.at[didx.at[pf, 3]], dsems[1]).wait()
    plsc.subcore_barrier()

    wt = acc_rows // NS

    @pl.when(c == 0)
    def _():
        pltpu.sync_copy(acc.at[pl.ds(s * wt, wt)], degp.at[0, pl.ds(s * wt, wt)])

    @pl.when(c == 1)
    def _():
        pltpu.sync_copy(acc.at[pl.ds(s * wt, wt)], degp.at[1, pl.ds(s * wt, wt)])


def _gather_body(hs0, hs1, src2, dst2, out_s, acc, rows, sidx, didx,
                 gsem0, gsem1, gsem2, gsem3, ssem0, isem):
    c = lax.axis_index("c")
    s = lax.axis_index("s")
    acc_rows = acc.shape[0]

    z16 = jnp.zeros((16,), jnp.float32)

    def zrow(r, _):
        rows[0, r, pl.ds(0, 16)] = z16
        rows[0, r, pl.ds(16, 16)] = z16
        return 0

    lax.fori_loop(0, 128, zrow, 0)

    zt = acc_rows // NS

    def zcopy(i, _):
        pltpu.sync_copy(rows.at[0], acc.at[pl.ds(s * zt + i * 128, 128)])
        return 0

    lax.fori_loop(0, zt // 128, zcopy, 0)
    plsc.subcore_barrier()

    rows_per_tile = src2.shape[0] // NS
    row0 = s * rows_per_tile
    ng = rows_per_tile // 8
    gsems = (gsem0, gsem1, gsem2, gsem3)

    def run(hs_ref):
        pltpu.sync_copy(src2.at[pl.ds(row0, 8)], sidx.at[0])
        pltpu.sync_copy(dst2.at[pl.ds(row0, 8)], didx.at[0])
        for j in range(3):
            pltpu.async_copy(hs_ref.at[sidx.at[0, j]], rows.at[j], gsems[j])

        def group(g, _):
            p = lax.rem(g, 2)
            q = 1 - p

            @pl.when(g + 1 < ng)
            def _():
                pltpu.async_copy(src2.at[pl.ds(row0 + (g + 1) * 8, 8)],
                                 sidx.at[q], isem)
                pltpu.async_copy(dst2.at[pl.ds(row0 + (g + 1) * 8, 8)],
                                 didx.at[q], isem)

            for j in range(8):
                jb = j & 3
                pltpu.make_async_copy(hs_ref.at[sidx.at[p, j]],
                                      rows.at[jb], gsems[j & 3]).wait()
                if j >= 1:
                    pltpu.make_async_copy(
                        rows.at[(j - 1) & 3],
                        acc.at[didx.at[p, j - 1]], ssem0).wait()
                else:
                    @pl.when(g > 0)
                    def _():
                        pltpu.make_async_copy(
                            rows.at[3],
                            acc.at[didx.at[lax.rem(g + 1, 2), 7]],
                            ssem0).wait()
                nsem = gsems[(j + 3) & 3]
                if j < 5:
                    pltpu.async_copy(hs_ref.at[sidx.at[p, j + 3]],
                                     rows.at[(j + 3) & 3], nsem)
                elif j == 5:
                    @pl.when(g + 1 < ng)
                    def _():
                        pltpu.make_async_copy(src2.at[pl.ds(0, 8)],
                                              sidx.at[q], isem).wait()
                        pltpu.make_async_copy(dst2.at[pl.ds(0, 8)],
                                              didx.at[q], isem).wait()
                        pltpu.async_copy(hs_ref.at[sidx.at[q, 0]],
                                         rows.at[(j + 3) & 3], nsem)
                else:
                    jn = j - 5
                    @pl.when(g + 1 < ng)
                    def _():
                        pltpu.async_copy(hs_ref.at[sidx.at[q, jn]],
                                         rows.at[(j + 3) & 3], nsem)
                pltpu.async_copy(rows.at[jb], acc.at[didx.at[p, j]],
                                 ssem0, add=True)
            return 0

        lax.fori_loop(0, ng, group, 0)
        pf = (ng - 1) % 2
        pltpu.make_async_copy(rows.at[3], acc.at[didx.at[pf, 7]],
                              ssem0).wait()

    @pl.when(c == 0)
    def _():
        run(hs0)

    @pl.when(c == 1)
    def _():
        run(hs1)

    plsc.subcore_barrier()
    wt = acc_rows // NS

    @pl.when(c == 0)
    def _():
        pltpu.sync_copy(acc.at[pl.ds(s * wt, wt)],
                        out_s.at[0, pl.ds(s * wt, wt)])

    @pl.when(c == 1)
    def _():
        pltpu.sync_copy(acc.at[pl.ds(s * wt, wt)],
                        out_s.at[1, pl.ds(s * wt, wt)])


def _proj_body(xp_ref, w0_ref, w1_ref, dp_ref, hs0_ref, hs1_ref):
    xb = xp_ref[...]
    d = dp_ref[...]
    hs0_ref[...] = d * jnp.dot(xb, w0_ref[...],
                               preferred_element_type=jnp.float32)
    hs1_ref[...] = d * jnp.dot(xb, w1_ref[...],
                               preferred_element_type=jnp.float32)


def _epi_body(s0_ref, s1_ref, h0_ref, h1_ref, dp_ref, b10_ref, b11_ref,
              w20_ref, w21_ref, b2p_ref, out_ref):
    d = dp_ref[...]
    t0 = jnp.maximum((s0_ref[...] + h0_ref[...]) * d + b10_ref[...], 0.0)
    t1 = jnp.maximum((s1_ref[...] + h1_ref[...]) * d + b11_ref[...], 0.0)
    out_ref[...] = (
        jnp.dot(t0, w20_ref[...], preferred_element_type=jnp.float32)
        + jnp.dot(t1, w21_ref[...], preferred_element_type=jnp.float32)
        + b2p_ref[...])


def kernel(x, edge_index, W1, b1, W2, b2):
    n = x.shape[0]
    e = edge_index.shape[1]
    d_hid = W1.shape[1]
    d_out = W2.shape[1]
    half = d_hid // 2

    group = LANES * NS * 8
    ep = ((e + group - 1) // group) * group
    acc_rows = ((n + 1 + NS * 128 - 1) // (NS * 128)) * (NS * 128)

    src = edge_index[0]
    dst = edge_index[1]
    pad = ep - e
    src_p = jnp.concatenate([src, jnp.zeros((pad,), jnp.int32)])
    dst_p = jnp.concatenate([dst, jnp.full((pad,), n, jnp.int32)])
    src2 = src_p.reshape(ep // LANES, LANES)
    dst2 = dst_p.reshape(ep // LANES, LANES)

    mesh = plsc.VectorSubcoreMesh(core_axis_name="c", subcore_axis_name="s")
    sc_params = pltpu.CompilerParams(use_tc_tiling_on_sc=False)

    degp = pl.kernel(
        _deg_body,
        out_type=jax.ShapeDtypeStruct((2, acc_rows), jnp.float32),
        mesh=mesh,
        scratch_types=[
            pltpu.VMEM_SHARED((acc_rows,), jnp.float32),
            pltpu.VMEM((acc_rows // NS,), jnp.float32),
            pltpu.VMEM((LANES,), jnp.float32),
            pltpu.VMEM((2, 4, LANES), jnp.int32),
            pltpu.SemaphoreType.DMA,
            pltpu.SemaphoreType.DMA,
            pltpu.SemaphoreType.DMA,
        ],
        compiler_params=sc_params,
    )(dst2)

    pb = 1024
    g = (n + pb - 1) // pb
    np_pad = g * pb
    prows = np_pad // 4

    deg = degp[0, :np_pad] + degp[1, :np_pad] + 1.0
    disp = jnp.repeat(lax.rsqrt(deg), half).reshape(prows, 128)

    x_p = x.reshape(n // 4, 4 * x.shape[1])
    eye4 = jnp.eye(4, dtype=jnp.float32)
    w1b0 = jnp.kron(eye4, W1[:, :half])
    w1b1 = jnp.kron(eye4, W1[:, half:])

    hsp0, hsp1 = pl.pallas_call(
        _proj_body,
        grid=(g,),
        in_specs=[
            pl.BlockSpec((pb // 4, 4 * x.shape[1]), lambda j: (j, 0)),
            pl.BlockSpec((4 * x.shape[1], 128), lambda j: (0, 0)),
            pl.BlockSpec((4 * x.shape[1], 128), lambda j: (0, 0)),
            pl.BlockSpec((pb // 4, 128), lambda j: (j, 0)),
        ],
        out_specs=[
            pl.BlockSpec((pb // 4, 128), lambda j: (j, 0)),
            pl.BlockSpec((pb // 4, 128), lambda j: (j, 0)),
        ],
        out_shape=[
            jax.ShapeDtypeStruct((prows, 128), jnp.float32),
            jax.ShapeDtypeStruct((prows, 128), jnp.float32),
        ],
    )(x_p, w1b0, w1b1, disp)

    hs0_lin = hsp0.reshape(np_pad, half)
    hs1_lin = hsp1.reshape(np_pad, half)

    s_agg = pl.kernel(
        _gather_body,
        out_type=jax.ShapeDtypeStruct((2, acc_rows, half), jnp.float32),
        mesh=mesh,
        scratch_types=[
            pltpu.VMEM_SHARED((acc_rows, half), jnp.float32),
            pltpu.VMEM((4, LANES, half), jnp.float32),
            pltpu.VMEM((2, 8, LANES), jnp.int32),
            pltpu.VMEM((2, 8, LANES), jnp.int32),
            pltpu.SemaphoreType.DMA,
            pltpu.SemaphoreType.DMA,
            pltpu.SemaphoreType.DMA,
            pltpu.SemaphoreType.DMA,
            pltpu.SemaphoreType.DMA,
            pltpu.SemaphoreType.DMA,
        ],
        compiler_params=sc_params,
    )(hs0_lin, hs1_lin, src2, dst2)

    s_lin = s_agg.reshape(2 * acc_rows * half // 128, 128)
    off1 = acc_rows * half // 128 // (pb // 4)

    w2b0 = jnp.kron(eye4, W2[:half, :])
    w2b1 = jnp.kron(eye4, W2[half:, :])
    b1p0 = jnp.tile(b1[:half], 4).reshape(1, 128)
    b1p1 = jnp.tile(b1[half:], 4).reshape(1, 128)
    b2p = jnp.tile(b2, 4).reshape(1, 4 * d_out)

    out_p = pl.pallas_call(
        _epi_body,
        grid=(g,),
        in_specs=[
            pl.BlockSpec((pb // 4, 128), lambda j: (j, 0)),
            pl.BlockSpec((pb // 4, 128), lambda j: (j + off1, 0)),
            pl.BlockSpec((pb // 4, 128), lambda j: (j, 0)),
            pl.BlockSpec((pb // 4, 128), lambda j: (j, 0)),
            pl.BlockSpec((pb // 4, 128), lambda j: (j, 0)),
            pl.BlockSpec((1, 128), lambda j: (0, 0)),
            pl.BlockSpec((1, 128), lambda j: (0, 0)),
            pl.BlockSpec((128, 4 * d_out), lambda j: (0, 0)),
            pl.BlockSpec((128, 4 * d_out), lambda j: (0, 0)),
            pl.BlockSpec((1, 4 * d_out), lambda j: (0, 0)),
        ],
        out_specs=pl.BlockSpec((pb // 4, 4 * d_out), lambda j: (j, 0)),
        out_shape=jax.ShapeDtypeStruct((prows, 4 * d_out), jnp.float32),
    )(s_lin, s_lin, hsp0, hsp1, disp, b1p0, b1p1, w2b0, w2b1, b2p)

    return out_p.reshape(np_pad, d_out)[:n]


# speedup vs baseline: 1.0083x; 1.0083x over previous
import functools

import jax
import jax.numpy as jnp
from jax import lax
from jax.experimental import pallas as pl
from jax.experimental.pallas import tpu as pltpu
from jax.experimental.pallas import tpu_sc as plsc

NC = 2
NS = 16
LANES = 128


def _deg_body(dst2, degp, acc, zbuf, ones, didx, sem, dsem0, dsem1):
    c = lax.axis_index("c")
    s = lax.axis_index("s")
    wid = s * NC + c
    acc_rows = acc.shape[0]
    zt = zbuf.shape[0]

    z16 = jnp.zeros((16,), jnp.float32)

    def zb(i, _):
        zbuf[pl.ds(i * 16, 16)] = z16
        return 0

    lax.fori_loop(0, zt // 16, zb, 0)

    def ob(i, _):
        ones[pl.ds(i * 16, 16)] = z16 + 1.0
        return 0

    lax.fori_loop(0, 128 // 16, ob, 0)

    pltpu.sync_copy(zbuf, acc.at[pl.ds(s * zt, zt)])
    plsc.subcore_barrier()

    rows_per_tile = dst2.shape[0] // (NC * NS)
    row0 = wid * rows_per_tile
    ng = rows_per_tile // 4

    pltpu.sync_copy(dst2.at[pl.ds(row0, 4)], didx.at[0])
    dsems = (dsem0, dsem1)

    def outer(g, _):
        p = lax.rem(g, 2)
        q = 1 - p

        @pl.when(g + 1 < ng)
        def _():
            pltpu.async_copy(dst2.at[pl.ds(row0 + (g + 1) * 4, 4)],
                             didx.at[q], sem)

        for j in range(4):
            if j >= 2:
                pltpu.make_async_copy(ones, acc.at[didx.at[p, j - 2]],
                                      dsems[j & 1]).wait()
            else:
                @pl.when(g > 0)
                def _():
                    pltpu.make_async_copy(
                        ones, acc.at[didx.at[lax.rem(g + 1, 2), j + 2]],
                        dsems[j & 1]).wait()
            pltpu.async_copy(ones, acc.at[didx.at[p, j]], dsems[j & 1],
                             add=True)

        @pl.when(g + 1 < ng)
        def _():
            pltpu.make_async_copy(dst2.at[pl.ds(0, 4)], didx.at[q],
                                  sem).wait()
        return 0

    lax.fori_loop(0, ng, outer, 0)
    pf = (ng - 1) % 2
    pltpu.make_async_copy(ones, acc.at[didx.at[pf, 2]], dsems[0]).wait()
    pltpu.make_async_copy(ones, acc.at[didx.at[pf, 3]], dsems[1]).wait()
    plsc.subcore_barrier()

    wt = acc_rows // NS

    @pl.when(c == 0)
    def _():
        pltpu.sync_copy(acc.at[pl.ds(s * wt, wt)], degp.at[0, pl.ds(s * wt, wt)])

    @pl.when(c == 1)
    def _():
        pltpu.sync_copy(acc.at[pl.ds(s * wt, wt)], degp.at[1, pl.ds(s * wt, wt)])


def _gather_body(hs0, hs1, src2, dst2, out_s, acc, rows, zbuf, sidx, didx,
                 gsem0, gsem1, gsem2, gsem3, ssem0, isem, zsem):
    c = lax.axis_index("c")
    s = lax.axis_index("s")
    acc_rows = acc.shape[0]

    z16 = jnp.zeros((16,), jnp.float32)

    def zrow(r, _):
        zbuf[r, pl.ds(0, 16)] = z16
        zbuf[r, pl.ds(16, 16)] = z16
        return 0

    lax.fori_loop(0, 128, zrow, 0)

    zt = acc_rows // NS

    def zcopy(i, _):
        pltpu.async_copy(zbuf, acc.at[pl.ds(s * zt + i * 128, 128)], zsem)
        return 0

    lax.fori_loop(0, zt // 128, zcopy, 0)

    rows_per_tile = src2.shape[0] // NS
    row0 = s * rows_per_tile
    ng = rows_per_tile // 8
    gsems = (gsem0, gsem1, gsem2, gsem3)

    def run(hs_ref):
        pltpu.sync_copy(src2.at[pl.ds(row0, 8)], sidx.at[0])
        pltpu.sync_copy(dst2.at[pl.ds(row0, 8)], didx.at[0])
        for j in range(3):
            pltpu.async_copy(hs_ref.at[sidx.at[0, j]], rows.at[j], gsems[j])

        def zdrain(i, _):
            pltpu.make_async_copy(
                zbuf, acc.at[pl.ds(s * zt + i * 128, 128)], zsem).wait()
            return 0

        lax.fori_loop(0, zt // 128, zdrain, 0)
        plsc.subcore_barrier()

        def group(g, _):
            p = lax.rem(g, 2)
            q = 1 - p

            @pl.when(g + 1 < ng)
            def _():
                pltpu.async_copy(src2.at[pl.ds(row0 + (g + 1) * 8, 8)],
                                 sidx.at[q], isem)
                pltpu.async_copy(dst2.at[pl.ds(row0 + (g + 1) * 8, 8)],
                                 didx.at[q], isem)

            for j in range(8):
                jb = j & 3
                pltpu.make_async_copy(hs_ref.at[sidx.at[p, j]],
                                      rows.at[jb], gsems[j & 3]).wait()
                if j >= 1:
                    pltpu.make_async_copy(
                        rows.at[(j - 1) & 3],
                        acc.at[didx.at[p, j - 1]], ssem0).wait()
                else:
                    @pl.when(g > 0)
                    def _():
                        pltpu.make_async_copy(
                            rows.at[3],
                            acc.at[didx.at[lax.rem(g + 1, 2), 7]],
                            ssem0).wait()
                nsem = gsems[(j + 3) & 3]
                if j < 5:
                    pltpu.async_copy(hs_ref.at[sidx.at[p, j + 3]],
                                     rows.at[(j + 3) & 3], nsem)
                elif j == 5:
                    @pl.when(g + 1 < ng)
                    def _():
                        pltpu.make_async_copy(src2.at[pl.ds(0, 8)],
                                              sidx.at[q], isem).wait()
                        pltpu.make_async_copy(dst2.at[pl.ds(0, 8)],
                                              didx.at[q], isem).wait()
                        pltpu.async_copy(hs_ref.at[sidx.at[q, 0]],
                                         rows.at[(j + 3) & 3], nsem)
                else:
                    jn = j - 5
                    @pl.when(g + 1 < ng)
                    def _():
                        pltpu.async_copy(hs_ref.at[sidx.at[q, jn]],
                                         rows.at[(j + 3) & 3], nsem)
                pltpu.async_copy(rows.at[jb], acc.at[didx.at[p, j]],
                                 ssem0, add=True)
            return 0

        lax.fori_loop(0, ng, group, 0)
        pf = (ng - 1) % 2
        pltpu.make_async_copy(rows.at[3], acc.at[didx.at[pf, 7]],
                              ssem0).wait()

    @pl.when(c == 0)
    def _():
        run(hs0)

    @pl.when(c == 1)
    def _():
        run(hs1)

    plsc.subcore_barrier()
    wt = acc_rows // NS

    @pl.when(c == 0)
    def _():
        pltpu.sync_copy(acc.at[pl.ds(s * wt, wt)],
                        out_s.at[0, pl.ds(s * wt, wt)])

    @pl.when(c == 1)
    def _():
        pltpu.sync_copy(acc.at[pl.ds(s * wt, wt)],
                        out_s.at[1, pl.ds(s * wt, wt)])


def _proj_body(xp_ref, w0_ref, w1_ref, dp_ref, hs0_ref, hs1_ref):
    xb = xp_ref[...]
    d = dp_ref[...]
    hs0_ref[...] = d * jnp.dot(xb, w0_ref[...],
                               preferred_element_type=jnp.float32)
    hs1_ref[...] = d * jnp.dot(xb, w1_ref[...],
                               preferred_element_type=jnp.float32)


def _epi_body(s0_ref, s1_ref, h0_ref, h1_ref, dp_ref, b10_ref, b11_ref,
              w20_ref, w21_ref, b2p_ref, out_ref):
    d = dp_ref[...]
    t0 = jnp.maximum((s0_ref[...] + h0_ref[...]) * d + b10_ref[...], 0.0)
    t1 = jnp.maximum((s1_ref[...] + h1_ref[...]) * d + b11_ref[...], 0.0)
    out_ref[...] = (
        jnp.dot(t0, w20_ref[...], preferred_element_type=jnp.float32)
        + jnp.dot(t1, w21_ref[...], preferred_element_type=jnp.float32)
        + b2p_ref[...])


def kernel(x, edge_index, W1, b1, W2, b2):
    n = x.shape[0]
    e = edge_index.shape[1]
    d_hid = W1.shape[1]
    d_out = W2.shape[1]
    half = d_hid // 2

    group = LANES * NS * 8
    ep = ((e + group - 1) // group) * group
    acc_rows = ((n + 1 + NS * 128 - 1) // (NS * 128)) * (NS * 128)

    src = edge_index[0]
    dst = edge_index[1]
    pad = ep - e
    src_p = jnp.concatenate([src, jnp.zeros((pad,), jnp.int32)])
    dst_p = jnp.concatenate([dst, jnp.full((pad,), n, jnp.int32)])
    src2 = src_p.reshape(ep // LANES, LANES)
    dst2 = dst_p.reshape(ep // LANES, LANES)

    mesh = plsc.VectorSubcoreMesh(core_axis_name="c", subcore_axis_name="s")
    sc_params = pltpu.CompilerParams(use_tc_tiling_on_sc=False)

    degp = pl.kernel(
        _deg_body,
        out_type=jax.ShapeDtypeStruct((2, acc_rows), jnp.float32),
        mesh=mesh,
        scratch_types=[
            pltpu.VMEM_SHARED((acc_rows,), jnp.float32),
            pltpu.VMEM((acc_rows // NS,), jnp.float32),
            pltpu.VMEM((LANES,), jnp.float32),
            pltpu.VMEM((2, 4, LANES), jnp.int32),
            pltpu.SemaphoreType.DMA,
            pltpu.SemaphoreType.DMA,
            pltpu.SemaphoreType.DMA,
        ],
        compiler_params=sc_params,
    )(dst2)

    pb = 1024
    g = (n + pb - 1) // pb
    np_pad = g * pb
    prows = np_pad // 4

    deg = degp[0, :np_pad] + degp[1, :np_pad] + 1.0
    disp = jnp.repeat(lax.rsqrt(deg), half).reshape(prows, 128)

    x_p = x.reshape(n // 4, 4 * x.shape[1])
    eye4 = jnp.eye(4, dtype=jnp.float32)
    w1b0 = jnp.kron(eye4, W1[:, :half])
    w1b1 = jnp.kron(eye4, W1[:, half:])

    hsp0, hsp1 = pl.pallas_call(
        _proj_body,
        grid=(g,),
        in_specs=[
            pl.BlockSpec((pb // 4, 4 * x.shape[1]), lambda j: (j, 0)),
            pl.BlockSpec((4 * x.shape[1], 128), lambda j: (0, 0)),
            pl.BlockSpec((4 * x.shape[1], 128), lambda j: (0, 0)),
            pl.BlockSpec((pb // 4, 128), lambda j: (j, 0)),
        ],
        out_specs=[
            pl.BlockSpec((pb // 4, 128), lambda j: (j, 0)),
            pl.BlockSpec((pb // 4, 128), lambda j: (j, 0)),
        ],
        out_shape=[
            jax.ShapeDtypeStruct((prows, 128), jnp.float32),
            jax.ShapeDtypeStruct((prows, 128), jnp.float32),
        ],
    )(x_p, w1b0, w1b1, disp)

    hs0_lin = hsp0.reshape(np_pad, half)
    hs1_lin = hsp1.reshape(np_pad, half)

    s_agg = pl.kernel(
        _gather_body,
        out_type=jax.ShapeDtypeStruct((2, acc_rows, half), jnp.float32),
        mesh=mesh,
        scratch_types=[
            pltpu.VMEM_SHARED((acc_rows, half), jnp.float32),
            pltpu.VMEM((4, LANES, half), jnp.float32),
            pltpu.VMEM((LANES, half), jnp.float32),
            pltpu.VMEM((2, 8, LANES), jnp.int32),
            pltpu.VMEM((2, 8, LANES), jnp.int32),
            pltpu.SemaphoreType.DMA,
            pltpu.SemaphoreType.DMA,
            pltpu.SemaphoreType.DMA,
            pltpu.SemaphoreType.DMA,
            pltpu.SemaphoreType.DMA,
            pltpu.SemaphoreType.DMA,
            pltpu.SemaphoreType.DMA,
        ],
        compiler_params=sc_params,
    )(hs0_lin, hs1_lin, src2, dst2)

    s_lin = s_agg.reshape(2 * acc_rows * half // 128, 128)
    off1 = acc_rows * half // 128 // (pb // 4)

    w2b0 = jnp.kron(eye4, W2[:half, :])
    w2b1 = jnp.kron(eye4, W2[half:, :])
    b1p0 = jnp.tile(b1[:half], 4).reshape(1, 128)
    b1p1 = jnp.tile(b1[half:], 4).reshape(1, 128)
    b2p = jnp.tile(b2, 4).reshape(1, 4 * d_out)

    out_p = pl.pallas_call(
        _epi_body,
        grid=(g,),
        in_specs=[
            pl.BlockSpec((pb // 4, 128), lambda j: (j, 0)),
            pl.BlockSpec((pb // 4, 128), lambda j: (j + off1, 0)),
            pl.BlockSpec((pb // 4, 128), lambda j: (j, 0)),
            pl.BlockSpec((pb // 4, 128), lambda j: (j, 0)),
            pl.BlockSpec((pb // 4, 128), lambda j: (j, 0)),
            pl.BlockSpec((1, 128), lambda j: (0, 0)),
            pl.BlockSpec((1, 128), lambda j: (0, 0)),
            pl.BlockSpec((128, 4 * d_out), lambda j: (0, 0)),
            pl.BlockSpec((128, 4 * d_out), lambda j: (0, 0)),
            pl.BlockSpec((1, 4 * d_out), lambda j: (0, 0)),
        ],
        out_specs=pl.BlockSpec((pb // 4, 4 * d_out), lambda j: (j, 0)),
        out_shape=jax.ShapeDtypeStruct((prows, 4 * d_out), jnp.float32),
    )(s_lin, s_lin, hsp0, hsp1, disp, b1p0, b1p1, w2b0, w2b1, b2p)

    return out_p.reshape(np_pad, d_out)[:n]
